# baseline (speedup 1.0000x reference)
import functools

import jax
import jax.numpy as jnp
from jax import lax
from jax.experimental import pallas as pl
from jax.experimental.pallas import tpu as pltpu
from jax.experimental.pallas import tpu_sc as plsc

_NC = 2
_NS = 16
_NW = _NC * _NS
_IDX_CHUNK = 128


def _make_sc_gather(B, D, NE, NR):
    b_per_w = B // _NW
    nch = b_per_w // _IDX_CHUNK
    mesh = plsc.VectorSubcoreMesh(
        core_axis_name="c", subcore_axis_name="s",
        num_cores=_NC, num_subcores=_NS)

    f32 = jnp.float32

    @functools.partial(
        pl.kernel,
        mesh=mesh,
        compiler_params=pltpu.CompilerParams(use_tc_tiling_on_sc=False),
        out_type=(
            jax.ShapeDtypeStruct((B, D), f32),
            jax.ShapeDtypeStruct((B, D), f32),
            jax.ShapeDtypeStruct((B, D), f32),
            jax.ShapeDtypeStruct((B, D), f32),
            jax.ShapeDtypeStruct((B,), f32),
            jax.ShapeDtypeStruct((B,), f32),
        ),
        scratch_types=[
            pltpu.VMEM((nch, _IDX_CHUNK), jnp.int32),
            pltpu.VMEM((nch, _IDX_CHUNK), jnp.int32),
            pltpu.VMEM((nch, _IDX_CHUNK), jnp.int32),
            pltpu.VMEM((b_per_w, D), f32),
            pltpu.VMEM((b_per_w, D), f32),
            pltpu.VMEM((b_per_w, D), f32),
            pltpu.VMEM((b_per_w, D), f32),
            pltpu.VMEM((b_per_w,), f32),
            pltpu.VMEM((b_per_w,), f32),
            pltpu.SemaphoreType.DMA,
        ],
    )
    def sc_gather(e1_h, rel_h, e2_h, Eh_h, rvh_h, Wu_h, bs_h, bo_h,
                  o_e1, o_e2, o_ru, o_rv, o_b1, o_b2,
                  i1_v, i2_v, ir_v, e1_v, e2_v, ru_v, rv_v, b1_v, b2_v,
                  sem):
        wid = lax.axis_index("s") * _NC + lax.axis_index("c")
        base = wid * b_per_w
        for j in range(nch):
            off = base + j * _IDX_CHUNK
            pltpu.sync_copy(e1_h.at[pl.ds(off, _IDX_CHUNK)], i1_v.at[j])
            pltpu.sync_copy(e2_h.at[pl.ds(off, _IDX_CHUNK)], i2_v.at[j])
            pltpu.sync_copy(rel_h.at[pl.ds(off, _IDX_CHUNK)], ir_v.at[j])
        copies = []
        for j in range(nch):
            dst = pl.ds(j * _IDX_CHUNK, _IDX_CHUNK)
            copies.append(pltpu.async_copy(Eh_h.at[i1_v.at[j]], e1_v.at[dst], sem))
            copies.append(pltpu.async_copy(Eh_h.at[i2_v.at[j]], e2_v.at[dst], sem))
            copies.append(pltpu.async_copy(Wu_h.at[ir_v.at[j]], ru_v.at[dst], sem))
            copies.append(pltpu.async_copy(rvh_h.at[ir_v.at[j]], rv_v.at[dst], sem))
            copies.append(pltpu.async_copy(bs_h.at[i1_v.at[j]], b1_v.at[dst], sem))
            copies.append(pltpu.async_copy(bo_h.at[i2_v.at[j]], b2_v.at[dst], sem))
        for c in copies:
            c.wait()
        out_slc = pl.ds(base, b_per_w)
        pltpu.sync_copy(e1_v, o_e1.at[out_slc])
        pltpu.sync_copy(e2_v, o_e2.at[out_slc])
        pltpu.sync_copy(ru_v, o_ru.at[out_slc])
        pltpu.sync_copy(rv_v, o_rv.at[out_slc])
        pltpu.sync_copy(b1_v, o_b1.at[out_slc])
        pltpu.sync_copy(b2_v, o_b2.at[out_slc])

    return sc_gather


def _artanh(x):
    return 0.5 * jnp.log((1.0 + x) / (1.0 - x))


def _sqnorm(x):
    return jnp.sum(x * x, axis=-1, keepdims=True)


def _tc_math_body(e1_ref, e2_ref, ru_ref, rv_ref, b1_ref, b2_ref, o_ref):
    e1 = e1_ref[...]
    e2 = e2_ref[...]
    ru = ru_ref[...]
    rv = rv_ref[...]

    def project(x):
        n = jnp.sqrt(_sqnorm(x))
        return jnp.where(n >= 1.0, x / (n - 1e-5), x)

    def p_log_map(v):
        normv = jnp.clip(jnp.sqrt(_sqnorm(v)), 1e-10, 1.0 - 1e-5)
        return _artanh(normv) * v / normv

    def p_exp_map(v):
        normv = jnp.clip(jnp.sqrt(_sqnorm(v)), 1e-10, None)
        return jnp.tanh(normv) * v / normv

    def p_sum(x, y):
        sqx = jnp.clip(_sqnorm(x), 0.0, 1.0 - 1e-5)
        sqy = jnp.clip(_sqnorm(y), 0.0, 1.0 - 1e-5)
        dxy = jnp.sum(x * y, axis=-1, keepdims=True)
        num = (1.0 + 2.0 * dxy + sqy) * x + (1.0 - sqx) * y
        den = 1.0 + 2.0 * dxy + sqx * sqy
        return num / den

    e1o = project(e1)
    e2o = project(e2)
    rvp = project(rv)
    e1w = p_log_map(e1o) * ru
    e1m = project(p_exp_map(e1w))
    e2m = project(p_sum(e2o, rvp))
    diff = p_sum(-e1m, e2m)
    nrm = jnp.clip(jnp.sqrt(_sqnorm(diff)[..., 0]), 1e-10, 1.0 - 1e-5)
    sqdist = (2.0 * _artanh(nrm)) ** 2
    o_ref[...] = -sqdist + b1_ref[...] + b2_ref[...]


def _tc_math(e1, e2, ru, rv, b1, b2, B, D, chunk=2048):
    grid = (B // chunk,)
    row_spec = pl.BlockSpec((chunk, D), lambda i: (i, 0))
    vec_spec = pl.BlockSpec((chunk,), lambda i: (i,))
    return pl.pallas_call(
        _tc_math_body,
        grid=grid,
        in_specs=[row_spec, row_spec, row_spec, row_spec, vec_spec, vec_spec],
        out_specs=vec_spec,
        out_shape=jax.ShapeDtypeStruct((B,), jnp.float32),
    )(e1, e2, ru, rv, b1, b2)


def kernel(entity1, relation, entity2, Eh, rvh, Wu, bs, bo):
    B = entity1.shape[0]
    NE, D = Eh.shape
    NR = rvh.shape[0]
    gat = _make_sc_gather(B, D, NE, NR)
    e1, e2, ru, rv, b1, b2 = gat(
        entity1.astype(jnp.int32), relation.astype(jnp.int32),
        entity2.astype(jnp.int32), Eh, rvh, Wu, bs, bo)
    return _tc_math(e1, e2, ru, rv, b1, b2, B, D)


# speedup vs baseline: 1.0813x; 1.0813x over previous
import functools

import jax
import jax.numpy as jnp
from jax import lax
from jax.experimental import pallas as pl
from jax.experimental.pallas import tpu as pltpu
from jax.experimental.pallas import tpu_sc as plsc

_NC = 2
_NS = 16
_NW = _NC * _NS
_L = 16
_IDX_CHUNK = 128


def _vfull(v):
    return jnp.full((_L,), v, jnp.float32)


def _vifull(v):
    return jnp.full((_L,), v, jnp.int32)


def _sqrt(s):
    i = plsc.bitcast(s, jnp.int32)
    y = plsc.bitcast(_vifull(0x5F3759DF) - lax.shift_right_arithmetic(i, _vifull(1)), jnp.float32)
    half, threehalf = _vfull(0.5), _vfull(1.5)
    for _ in range(3):
        y = y * (threehalf - half * s * y * y)
    return s * y


def _tanh_pos(x):
    t = jnp.exp(_vfull(-2.0) * x)
    big = (_vfull(1.0) - t) / (_vfull(1.0) + t)
    x2 = x * x
    ser = x * (_vfull(1.0) + x2 * (_vfull(-1.0 / 3.0) + x2 * _vfull(2.0 / 15.0)))
    return jnp.where(x < _vfull(0.04), ser, big)


def _log_ge1(x):
    i = plsc.bitcast(x, jnp.int32)
    e = (lax.shift_right_arithmetic(i, _vifull(23)) - _vifull(127)).astype(jnp.float32)
    m = plsc.bitcast(
        jnp.bitwise_or(jnp.bitwise_and(i, _vifull(0x007FFFFF)), _vifull(0x3F800000)),
        jnp.float32)
    big = m > _vfull(1.41421356)
    m = jnp.where(big, _vfull(0.5) * m, m)
    e = jnp.where(big, e + _vfull(1.0), e)
    t = (m - _vfull(1.0)) / (m + _vfull(1.0))
    t2 = t * t
    p = t * (_vfull(2.0) + t2 * (_vfull(2.0 / 3.0) + t2 * (
        _vfull(2.0 / 5.0) + t2 * (_vfull(2.0 / 7.0) + t2 * _vfull(2.0 / 9.0)))))
    return e * _vfull(0.6931471805599453) + p


def _artanh(y):
    big = _vfull(0.5) * _log_ge1((_vfull(1.0) + y) / (_vfull(1.0) - y))
    y2 = y * y
    ser = y * (_vfull(1.0) + y2 * (_vfull(1.0 / 3.0) + y2 * _vfull(0.2)))
    return jnp.where(y < _vfull(0.03), ser, big)


def _proj_scale(n):
    return jnp.where(n >= _vfull(1.0), _vfull(1.0) / (n - _vfull(1e-5)), _vfull(1.0))


def _score16(P, Q, Rr, W2, A, C, Dd, b1v, b2v):
    one = _vfull(1.0)
    two = _vfull(2.0)
    lim = _vfull(1.0 - 1e-5)
    tiny = _vfull(1e-10)

    s1 = _proj_scale(_sqrt(P))
    s2 = _proj_scale(_sqrt(Q))
    s3 = _proj_scale(_sqrt(Rr))

    n1 = jnp.clip(_sqrt(P) * s1, tiny, lim)
    fl = _artanh(n1) / n1
    nW = s1 * fl * _sqrt(W2)
    nWc = jnp.maximum(nW, tiny)
    fe = _tanh_pos(nWc) / nWc
    s41 = _proj_scale(fe * nW)
    c1 = s1 * fl * fe * s41

    sqx = jnp.minimum(s2 * s2 * Q, lim)
    sqy = jnp.minimum(s3 * s3 * Rr, lim)
    dxy = s2 * s3 * Dd
    aa = one + two * dxy + sqy
    bf = one - sqx
    r = one / (one + two * dxy + sqx * sqy)
    al = r * aa * s2
    be = r * bf * s3
    n2m = _sqrt(jnp.maximum(al * al * Q + two * al * be * Dd + be * be * Rr, _vfull(0.0)))
    s42 = _proj_scale(n2m)
    al = s42 * al
    be = s42 * be

    U = c1 * c1 * W2
    V = al * al * Q + two * al * be * Dd + be * be * Rr
    G = -c1 * (al * A + be * C)
    Uc = jnp.minimum(U, lim)
    Vc = jnp.minimum(V, lim)
    a2 = one + two * G + Vc
    b2f = one - Uc
    den2 = one + two * G + Uc * Vc
    sn2 = jnp.maximum(a2 * a2 * U + two * a2 * b2f * G + b2f * b2f * V, _vfull(0.0))
    nrm = jnp.clip(_sqrt(sn2) / jnp.abs(den2), tiny, lim)
    at = _artanh(nrm)
    return -(two * at) * (two * at) + b1v + b2v


def _make_sc_kernel(B, D, NE, NR):
    b_per_w = B // _NW
    nch = b_per_w // _IDX_CHUNK
    ngrp = b_per_w // _L
    mesh = plsc.VectorSubcoreMesh(
        core_axis_name="c", subcore_axis_name="s",
        num_cores=_NC, num_subcores=_NS)
    f32 = jnp.float32

    @functools.partial(
        pl.kernel,
        mesh=mesh,
        compiler_params=pltpu.CompilerParams(
            use_tc_tiling_on_sc=False, needs_layout_passes=False),
        out_type=jax.ShapeDtypeStruct((B,), f32),
        scratch_types=[
            pltpu.VMEM((nch, _IDX_CHUNK), jnp.int32),
            pltpu.VMEM((nch, _IDX_CHUNK), jnp.int32),
            pltpu.VMEM((nch, _IDX_CHUNK), jnp.int32),
            pltpu.VMEM((b_per_w, D), f32),
            pltpu.VMEM((b_per_w, D), f32),
            pltpu.VMEM((b_per_w, D), f32),
            pltpu.VMEM((b_per_w, D), f32),
            pltpu.VMEM((b_per_w,), f32),
            pltpu.VMEM((b_per_w,), f32),
            pltpu.VMEM((b_per_w,), f32),
            pltpu.SemaphoreType.DMA,
        ],
    )
    def sc_kernel(e1_h, rel_h, e2_h, Eh_h, rvh_h, Wu_h, bs_h, bo_h,
                  out_h,
                  i1_v, i2_v, ir_v, e1_v, e2_v, ru_v, rv_v, b1_v, b2_v,
                  out_v, sem):
        wid = lax.axis_index("s") * _NC + lax.axis_index("c")
        base = wid * b_per_w
        for j in range(nch):
            off = base + j * _IDX_CHUNK
            pltpu.sync_copy(e1_h.at[pl.ds(off, _IDX_CHUNK)], i1_v.at[j])
            pltpu.sync_copy(e2_h.at[pl.ds(off, _IDX_CHUNK)], i2_v.at[j])
            pltpu.sync_copy(rel_h.at[pl.ds(off, _IDX_CHUNK)], ir_v.at[j])
        copies = []
        for j in range(nch):
            dst = pl.ds(j * _IDX_CHUNK, _IDX_CHUNK)
            copies.append(pltpu.async_copy(Eh_h.at[i1_v.at[j]], e1_v.at[dst], sem))
            copies.append(pltpu.async_copy(Eh_h.at[i2_v.at[j]], e2_v.at[dst], sem))
            copies.append(pltpu.async_copy(Wu_h.at[ir_v.at[j]], ru_v.at[dst], sem))
            copies.append(pltpu.async_copy(rvh_h.at[ir_v.at[j]], rv_v.at[dst], sem))
            copies.append(pltpu.async_copy(bs_h.at[i1_v.at[j]], b1_v.at[dst], sem))
            copies.append(pltpu.async_copy(bo_h.at[i2_v.at[j]], b2_v.at[dst], sem))
        for c in copies:
            c.wait()

        iota = lax.iota(jnp.int32, _L)

        def group(g, carry):
            rows = g * _L + iota
            zero = _vfull(0.0)
            P = Q = Rr = W2 = A = C = Dd = zero
            for d in range(D):
                dsplat = _vifull(d)
                a1 = plsc.load_gather(e1_v, [rows, dsplat])
                a2 = plsc.load_gather(e2_v, [rows, dsplat])
                aru = plsc.load_gather(ru_v, [rows, dsplat])
                arv = plsc.load_gather(rv_v, [rows, dsplat])
                w = a1 * aru
                P = P + a1 * a1
                Q = Q + a2 * a2
                Rr = Rr + arv * arv
                W2 = W2 + w * w
                A = A + w * a2
                C = C + w * arv
                Dd = Dd + a2 * arv
            b1v = b1_v[pl.ds(g * _L, _L)]
            b2v = b2_v[pl.ds(g * _L, _L)]
            out_v[pl.ds(g * _L, _L)] = _score16(P, Q, Rr, W2, A, C, Dd, b1v, b2v)
            return carry

        lax.fori_loop(0, ngrp, group, 0)
        pltpu.sync_copy(out_v, out_h.at[pl.ds(base, b_per_w)])

    return sc_kernel


def kernel(entity1, relation, entity2, Eh, rvh, Wu, bs, bo):
    B = entity1.shape[0]
    NE, D = Eh.shape
    NR = rvh.shape[0]
    k = _make_sc_kernel(B, D, NE, NR)
    return k(entity1.astype(jnp.int32), relation.astype(jnp.int32),
             entity2.astype(jnp.int32), Eh, rvh, Wu, bs, bo)
